# initial kernel scaffold (unmeasured)
import jax
import jax.numpy as jnp
from jax import lax
from jax.experimental import pallas as pl
from jax.experimental.pallas import tpu as pltpu


def kernel(
    x,
):
    def body(*refs):
        pass

    out_shape = jax.ShapeDtypeStruct(..., jnp.float32)
    return pl.pallas_call(body, out_shape=out_shape)(...)


# baseline (device time: 13819 ns/iter reference)
import jax
import jax.numpy as jnp
from jax import lax
from jax.experimental import pallas as pl
from jax.experimental.pallas import tpu as pltpu


def kernel(x):
    m, n = x.shape
    half = m // 2

    def body(x_ref, out_ref, send_buf, recv_buf, send_sems, recv_sems):
        my_x = lax.axis_index("x")
        my_y = lax.axis_index("y")

        barrier = pltpu.get_barrier_semaphore()
        pl.semaphore_signal(
            barrier, inc=1, device_id=(1 - my_x, my_y),
            device_id_type=pl.DeviceIdType.MESH,
        )
        pl.semaphore_signal(
            barrier, inc=1, device_id=(my_x, 1 - my_y),
            device_id_type=pl.DeviceIdType.MESH,
        )
        pl.semaphore_wait(barrier, 2)

        row0 = my_y * half

        send_buf[...] = x_ref[pl.ds(row0, half), :].astype(jnp.bfloat16)
        p1 = pltpu.make_async_remote_copy(
            src_ref=send_buf,
            dst_ref=recv_buf,
            send_sem=send_sems.at[0],
            recv_sem=recv_sems.at[0],
            device_id=(1 - my_x, my_y),
            device_id_type=pl.DeviceIdType.MESH,
        )
        p1.start()
        p1.wait()

        out_ref[pl.ds(row0, half), :] = (
            x_ref[pl.ds(row0, half), :] + recv_buf[...].astype(jnp.float32)
        ).astype(jnp.bfloat16)

        p2 = pltpu.make_async_remote_copy(
            src_ref=out_ref.at[pl.ds(row0, half)],
            dst_ref=out_ref.at[pl.ds(row0, half)],
            send_sem=send_sems.at[1],
            recv_sem=recv_sems.at[1],
            device_id=(my_x, 1 - my_y),
            device_id_type=pl.DeviceIdType.MESH,
        )
        p2.start()
        p2.wait()

    return pl.pallas_call(
        body,
        out_shape=jax.ShapeDtypeStruct((m, n), jnp.bfloat16),
        in_specs=[pl.BlockSpec(memory_space=pltpu.VMEM)],
        out_specs=pl.BlockSpec(memory_space=pltpu.VMEM),
        scratch_shapes=[
            pltpu.VMEM((half, n), jnp.bfloat16),
            pltpu.VMEM((half, n), jnp.bfloat16),
            pltpu.SemaphoreType.DMA((2,)),
            pltpu.SemaphoreType.DMA((2,)),
        ],
        compiler_params=pltpu.CompilerParams(collective_id=0),
    )(x)


# device time: 11763 ns/iter; 1.1748x vs baseline; 1.1748x over previous
import jax
import jax.numpy as jnp
from jax import lax
from jax.experimental import pallas as pl
from jax.experimental.pallas import tpu as pltpu


C = 4


def kernel(x):
    m, n = x.shape
    half = m // 2
    rows = half // C

    def body(x_ref, out_ref, send_buf, recv_buf,
             p1_send, p1_recv, p2_send, p2_recv):
        my_x = lax.axis_index("x")
        my_y = lax.axis_index("y")

        barrier = pltpu.get_barrier_semaphore()
        pl.semaphore_signal(
            barrier, inc=1, device_id=(1 - my_x, my_y),
            device_id_type=pl.DeviceIdType.MESH,
        )
        pl.semaphore_signal(
            barrier, inc=1, device_id=(my_x, 1 - my_y),
            device_id_type=pl.DeviceIdType.MESH,
        )
        pl.semaphore_wait(barrier, 2)

        row0 = my_y * half

        p1 = []
        for c in range(C):
            send_buf[pl.ds(c * rows, rows), :] = (
                x_ref[pl.ds(row0 + c * rows, rows), :].astype(jnp.bfloat16)
            )
            r = pltpu.make_async_remote_copy(
                src_ref=send_buf.at[pl.ds(c * rows, rows)],
                dst_ref=recv_buf.at[pl.ds(c * rows, rows)],
                send_sem=p1_send.at[c],
                recv_sem=p1_recv.at[c],
                device_id=(1 - my_x, my_y),
                device_id_type=pl.DeviceIdType.MESH,
            )
            r.start()
            p1.append(r)

        p2 = []
        for c in range(C):
            p1[c].wait_recv()
            out_ref[pl.ds(row0 + c * rows, rows), :] = (
                send_buf[pl.ds(c * rows, rows), :]
                + recv_buf[pl.ds(c * rows, rows), :]
            )
            r = pltpu.make_async_remote_copy(
                src_ref=out_ref.at[pl.ds(row0 + c * rows, rows)],
                dst_ref=out_ref.at[pl.ds(row0 + c * rows, rows)],
                send_sem=p2_send.at[c],
                recv_sem=p2_recv.at[c],
                device_id=(my_x, 1 - my_y),
                device_id_type=pl.DeviceIdType.MESH,
            )
            r.start()
            p2.append(r)

        for c in range(C):
            p2[c].wait_recv()
        for c in range(C):
            p1[c].wait_send()
            p2[c].wait_send()

    return pl.pallas_call(
        body,
        out_shape=jax.ShapeDtypeStruct((m, n), jnp.bfloat16),
        in_specs=[pl.BlockSpec(memory_space=pltpu.VMEM)],
        out_specs=pl.BlockSpec(memory_space=pltpu.VMEM),
        scratch_shapes=[
            pltpu.VMEM((half, n), jnp.bfloat16),
            pltpu.VMEM((half, n), jnp.bfloat16),
            pltpu.SemaphoreType.DMA((C,)),
            pltpu.SemaphoreType.DMA((C,)),
            pltpu.SemaphoreType.DMA((C,)),
            pltpu.SemaphoreType.DMA((C,)),
        ],
        compiler_params=pltpu.CompilerParams(collective_id=0),
    )(x)


# device time: 11557 ns/iter; 1.1957x vs baseline; 1.0178x over previous
import jax
import jax.numpy as jnp
from jax import lax
from jax.experimental import pallas as pl
from jax.experimental.pallas import tpu as pltpu

C = 8


def kernel(x):
    m, n = x.shape
    half = m // 2
    rows = half // C

    def body(x_ref, out_ref, send_buf, recv_buf,
             p1_send, p1_recv, p2_send, p2_recv):
        my_x = lax.axis_index("x")
        my_y = lax.axis_index("y")

        barrier = pltpu.get_barrier_semaphore()
        pl.semaphore_signal(
            barrier, inc=1, device_id=(1 - my_x, my_y),
            device_id_type=pl.DeviceIdType.MESH,
        )
        pl.semaphore_signal(
            barrier, inc=1, device_id=(my_x, 1 - my_y),
            device_id_type=pl.DeviceIdType.MESH,
        )

        def run(y):
            row0 = y * half

            send_buf[...] = x_ref[row0:row0 + half, :].astype(jnp.bfloat16)

            pl.semaphore_wait(barrier, 2)

            p1 = []
            for c in range(C):
                r = pltpu.make_async_remote_copy(
                    src_ref=send_buf.at[c * rows:(c + 1) * rows],
                    dst_ref=recv_buf.at[c * rows:(c + 1) * rows],
                    send_sem=p1_send.at[c],
                    recv_sem=p1_recv.at[c],
                    device_id=(1 - my_x, my_y),
                    device_id_type=pl.DeviceIdType.MESH,
                )
                r.start()
                p1.append(r)

            p2 = []
            for c in range(C):
                p1[c].wait_recv()
                out_ref[row0 + c * rows:row0 + (c + 1) * rows, :] = (
                    send_buf[c * rows:(c + 1) * rows, :]
                    + recv_buf[c * rows:(c + 1) * rows, :]
                )
                r = pltpu.make_async_remote_copy(
                    src_ref=out_ref.at[row0 + c * rows:row0 + (c + 1) * rows],
                    dst_ref=out_ref.at[row0 + c * rows:row0 + (c + 1) * rows],
                    send_sem=p2_send.at[c],
                    recv_sem=p2_recv.at[c],
                    device_id=(my_x, 1 - my_y),
                    device_id_type=pl.DeviceIdType.MESH,
                )
                r.start()
                p2.append(r)

            for c in range(C):
                p2[c].wait_recv()
            for c in range(C):
                p1[c].wait_send()
                p2[c].wait_send()

        @pl.when(my_y == 0)
        def _():
            run(0)

        @pl.when(my_y == 1)
        def _():
            run(1)

    return pl.pallas_call(
        body,
        out_shape=jax.ShapeDtypeStruct((m, n), jnp.bfloat16),
        in_specs=[pl.BlockSpec(memory_space=pltpu.VMEM)],
        out_specs=pl.BlockSpec(memory_space=pltpu.VMEM),
        scratch_shapes=[
            pltpu.VMEM((half, n), jnp.bfloat16),
            pltpu.VMEM((half, n), jnp.bfloat16),
            pltpu.SemaphoreType.DMA((C,)),
            pltpu.SemaphoreType.DMA((C,)),
            pltpu.SemaphoreType.DMA((C,)),
            pltpu.SemaphoreType.DMA((C,)),
        ],
        compiler_params=pltpu.CompilerParams(collective_id=0),
    )(x)
